# Initial kernel scaffold; baseline (speedup 1.0000x reference)
import jax
import jax.numpy as jnp
from jax.experimental import pallas as pl


def kernel(x, conv_w, conv_b):
    raise NotImplementedError("write your pallas kernel here")


# speedup vs baseline: 110.0296x; 110.0296x over previous
import functools

import numpy as np

import jax
import jax.numpy as jnp
from jax.experimental import pallas as pl
from jax.experimental.pallas import tpu as pltpu

_KS = 7
_PD = 3


@functools.lru_cache(maxsize=None)
def _onehot_taps(n, k):
    src = np.arange(n).repeat(n)
    dst = np.tile(np.arange(n), n)
    d = src - dst + _PD
    oh = np.zeros((n * n, k), np.float32)
    ok = (d >= 0) & (d < k)
    oh[np.nonzero(ok)[0], d[ok]] = 1.0
    return jnp.asarray(oh)


def _conv_matrix(w2d, height, width):
    a = _onehot_taps(height, _KS) @ w2d.astype(jnp.float32)
    m = a @ _onehot_taps(width, _KS).T
    m = m.reshape(height, height, width, width)
    return m.transpose(0, 2, 1, 3).reshape(height * width, height * width)


def _gate_mul_body(x_ref, g_ref, b_ref, o_ref, *, sub, cmul):
    Bt, C, S = x_ref.shape
    f32 = jnp.float32

    mx_parts, sm_parts = [], []
    for s0 in range(0, Bt, sub):
        xs = x_ref[s0:s0 + sub]
        mx_parts.append(jnp.max(xs, axis=1))
        sm_parts.append(jnp.sum(xs, axis=1))
    mx = jnp.concatenate(mx_parts, axis=0) if len(mx_parts) > 1 else mx_parts[0]
    sm = jnp.concatenate(sm_parts, axis=0) if len(sm_parts) > 1 else sm_parts[0]

    pooled = jnp.concatenate([mx, sm], axis=-1)
    y = jnp.dot(pooled, g_ref[...], preferred_element_type=f32) + b_ref[0]
    gate = jax.nn.sigmoid(y)

    for c0 in range(0, C, cmul):
        o_ref[:, c0:c0 + cmul, :] = x_ref[:, c0:c0 + cmul, :] * gate[:, None, :]


def kernel(x, conv_w, conv_b):
    B, C, H, W = x.shape
    S = H * W

    g = jnp.concatenate(
        [_conv_matrix(conv_w[0, 0], H, W),
         _conv_matrix(conv_w[0, 1], H, W) * (1.0 / C)], axis=0)
    b = conv_b.reshape(1).astype(jnp.float32)

    x_flat = x.reshape(B, C, S)

    bt = 32
    while B % bt:
        bt //= 2
    sub = min(8, bt)
    cmul = 8 if C % 8 == 0 else C

    body = functools.partial(_gate_mul_body, sub=sub, cmul=cmul)
    out = pl.pallas_call(
        body,
        out_shape=jax.ShapeDtypeStruct((B, C, S), x.dtype),
        grid=(B // bt,),
        in_specs=[
            pl.BlockSpec((bt, C, S), lambda i: (i, 0, 0)),
            pl.BlockSpec((2 * S, S), lambda i: (0, 0)),
            pl.BlockSpec(memory_space=pltpu.MemorySpace.SMEM),
        ],
        out_specs=pl.BlockSpec((bt, C, S), lambda i: (i, 0, 0)),
        compiler_params=pltpu.CompilerParams(
            dimension_semantics=("parallel",),
            vmem_limit_bytes=int(56 << 20),
        ),
    )(x_flat, g, b)

    return out.reshape(B, C, H, W)


# speedup vs baseline: 116.7135x; 1.0607x over previous
import functools

import numpy as np

import jax
import jax.numpy as jnp
from jax.experimental import pallas as pl
from jax.experimental.pallas import tpu as pltpu

_KS = 7
_PD = 3


@functools.lru_cache(maxsize=None)
def _onehot_taps(n, k):
    src = np.arange(n).repeat(n)
    dst = np.tile(np.arange(n), n)
    d = src - dst + _PD
    oh = np.zeros((n * n, k), np.float32)
    ok = (d >= 0) & (d < k)
    oh[np.nonzero(ok)[0], d[ok]] = 1.0
    return jnp.asarray(oh)


def _conv_matrix(w2d, height, width):
    a = _onehot_taps(height, _KS) @ w2d.astype(jnp.float32)
    m = a @ _onehot_taps(width, _KS).T
    m = m.reshape(height, height, width, width)
    return m.transpose(0, 2, 1, 3).reshape(height * width, height * width)


def _gate_mul_body(x_ref, g0_ref, g1_ref, b_ref, o_ref, *, sub, cmul):
    Bt, C, S = x_ref.shape
    f32 = jnp.float32

    mx_parts, sm_parts = [], []
    for s0 in range(0, Bt, sub):
        xs = x_ref[s0:s0 + sub]
        mx_parts.append(jnp.max(xs, axis=1))
        sm_parts.append(jnp.sum(xs, axis=1))
    mx = jnp.concatenate(mx_parts, axis=0) if len(mx_parts) > 1 else mx_parts[0]
    sm = jnp.concatenate(sm_parts, axis=0) if len(sm_parts) > 1 else sm_parts[0]

    y = (jnp.dot(mx.astype(jnp.bfloat16), g0_ref[...], preferred_element_type=f32)
         + jnp.dot(sm.astype(jnp.bfloat16), g1_ref[...], preferred_element_type=f32)
         + b_ref[0])
    gate = jax.nn.sigmoid(y)

    for c0 in range(0, C, cmul):
        o_ref[:, c0:c0 + cmul, :] = x_ref[:, c0:c0 + cmul, :] * gate[:, None, :]


def kernel(x, conv_w, conv_b):
    B, C, H, W = x.shape
    S = H * W

    g0 = _conv_matrix(conv_w[0, 0], H, W).astype(jnp.bfloat16)
    g1 = (_conv_matrix(conv_w[0, 1], H, W) * (1.0 / C)).astype(jnp.bfloat16)
    b = conv_b.reshape(1).astype(jnp.float32)

    x_flat = x.reshape(B, C, S)

    bt = 32
    while B % bt:
        bt //= 2
    sub = min(8, bt)
    cmul = 8 if C % 8 == 0 else C

    body = functools.partial(_gate_mul_body, sub=sub, cmul=cmul)
    out = pl.pallas_call(
        body,
        out_shape=jax.ShapeDtypeStruct((B, C, S), x.dtype),
        grid=(B // bt,),
        in_specs=[
            pl.BlockSpec((bt, C, S), lambda i: (i, 0, 0)),
            pl.BlockSpec((S, S), lambda i: (0, 0)),
            pl.BlockSpec((S, S), lambda i: (0, 0)),
            pl.BlockSpec(memory_space=pltpu.MemorySpace.SMEM),
        ],
        out_specs=pl.BlockSpec((bt, C, S), lambda i: (i, 0, 0)),
        compiler_params=pltpu.CompilerParams(
            dimension_semantics=("parallel",),
            vmem_limit_bytes=int(56 << 20),
        ),
    )(x_flat, g0, g1, b)

    return out.reshape(B, C, H, W)


# speedup vs baseline: 140.4136x; 1.2031x over previous
import functools

import jax
import jax.numpy as jnp
from jax.experimental import pallas as pl
from jax.experimental.pallas import tpu as pltpu

_KS = 7
_PD = 3


def _conv7x7(p, w_ref, w_base, th, tw, height, width):
    S = height * width
    cols = [None] * _KS
    for k in range(_KS):
        dh = k - _PD
        ph = pltpu.roll(p, (-dh * width) % S, 1)
        if dh:
            ph = jnp.where((th + dh >= 0) & (th + dh < height), ph, 0.0)
        for l in range(_KS):
            t = ph * w_ref[w_base + k * _KS + l]
            cols[l] = t if cols[l] is None else cols[l] + t
    y = None
    for l in range(_KS):
        dw = l - _PD
        yl = pltpu.roll(cols[l], (-dw) % S, 1)
        if dw:
            yl = jnp.where((tw + dw >= 0) & (tw + dw < width), yl, 0.0)
        y = yl if y is None else y + yl
    return y


def _body(x_ref, w_ref, o_ref, *, sub, cmul, height, width):
    Bt, C, S = x_ref.shape
    f32 = jnp.float32
    lane = jax.lax.broadcasted_iota(jnp.int32, (sub, S), 1)
    th = lane // width
    tw = lane - th * width

    gates = []
    for s0 in range(0, Bt, sub):
        xs = x_ref[s0:s0 + sub]
        mx = jnp.max(xs, axis=1)
        sm = jnp.sum(xs, axis=1) * (1.0 / C)
        y = (_conv7x7(mx, w_ref, 0, th, tw, height, width)
             + _conv7x7(sm, w_ref, _KS * _KS, th, tw, height, width)
             + w_ref[2 * _KS * _KS])
        gates.append(jax.nn.sigmoid(y))
    gate = jnp.concatenate(gates, axis=0) if len(gates) > 1 else gates[0]

    for c0 in range(0, C, cmul):
        o_ref[:, c0:c0 + cmul, :] = x_ref[:, c0:c0 + cmul, :] * gate[:, None, :]


def kernel(x, conv_w, conv_b):
    B, C, H, W = x.shape
    S = H * W

    wb = jnp.concatenate([conv_w.reshape(2 * _KS * _KS),
                          conv_b.reshape(1)]).astype(jnp.float32)

    x_flat = x.reshape(B, C, S)

    bt = 32
    while B % bt:
        bt //= 2
    sub = min(8, bt)
    cmul = 8 if C % 8 == 0 else C

    body = functools.partial(_body, sub=sub, cmul=cmul, height=H, width=W)
    out = pl.pallas_call(
        body,
        out_shape=jax.ShapeDtypeStruct((B, C, S), x.dtype),
        grid=(B // bt,),
        in_specs=[
            pl.BlockSpec((bt, C, S), lambda i: (i, 0, 0)),
            pl.BlockSpec(memory_space=pltpu.MemorySpace.SMEM),
        ],
        out_specs=pl.BlockSpec((bt, C, S), lambda i: (i, 0, 0)),
        compiler_params=pltpu.CompilerParams(
            dimension_semantics=("parallel",),
            vmem_limit_bytes=int(56 << 20),
        ),
    )(x_flat, wb)

    return out.reshape(B, C, H, W)
